# baseline (speedup 1.0000x reference)
import functools

import jax
import jax.numpy as jnp
from jax import lax
from jax.experimental import pallas as pl
from jax.experimental.pallas import tpu as pltpu
from jax.experimental.pallas import tpu_sc as plsc

SEQ = 2048
ROWS = 128
NUM_REGIONS = 32768
HIDDEN = 64
TOP_K = 8
CHUNK = 64
LANES = 16
SLICES = NUM_REGIONS // LANES

NEG = float("-inf")
BIG = 3.0e38


def _mlp_body(periph_ref, imu_ref, traj_ref, w1p_ref, w1i_ref, w1t_ref,
              b1_ref, w2_ref, b2_ref, scores_ref, accp, acci, acct):
    g = pl.program_id(0)

    @pl.when(g == 0)
    def _init():
        accp[...] = jnp.zeros_like(accp)
        acci[...] = jnp.zeros_like(acci)
        acct[...] = jnp.zeros_like(acct)

    accp[...] += jnp.sum(periph_ref[...], axis=0)
    acci[...] += jnp.sum(imu_ref[...].reshape(CHUNK, 16, 128), axis=0)
    acct[...] += jnp.sum(traj_ref[...].reshape(CHUNK, 32, 128), axis=0)

    @pl.when(g == (SEQ // CHUNK) - 1)
    def _final():
        inv = jnp.float32(1.0 / SEQ)
        xi = (acci[...] * inv).reshape(ROWS, 16)
        xt = (acct[...] * inv).reshape(ROWS, 32)
        pre = (accp[...] * inv) @ w1p_ref[...]
        pre += xi @ w1i_ref[...]
        pre += xt @ w1t_ref[...]
        h = jnp.maximum(pre + b1_ref[...], 0.0)
        scores_ref[...] = h @ w2_ref[...] + b2_ref[...]


def _scores_tc(periph_seq, imu3, traj3, w1p, w1i, w1t, b1r, W2, b2r):
    nsteps = SEQ // CHUNK
    return pl.pallas_call(
        _mlp_body,
        grid=(nsteps,),
        in_specs=[
            pl.BlockSpec((CHUNK, ROWS, 128), lambda g: (g, 0, 0)),
            pl.BlockSpec((CHUNK * 2048,), lambda g: (g,)),
            pl.BlockSpec((CHUNK * 4096,), lambda g: (g,)),
            pl.BlockSpec((128, HIDDEN), lambda g: (0, 0)),
            pl.BlockSpec((16, HIDDEN), lambda g: (0, 0)),
            pl.BlockSpec((32, HIDDEN), lambda g: (0, 0)),
            pl.BlockSpec((1, HIDDEN), lambda g: (0, 0)),
            pl.BlockSpec((HIDDEN, NUM_REGIONS), lambda g: (0, 0)),
            pl.BlockSpec((1, NUM_REGIONS), lambda g: (0, 0)),
        ],
        out_specs=pl.BlockSpec((ROWS, NUM_REGIONS), lambda g: (0, 0)),
        out_shape=jax.ShapeDtypeStruct((ROWS, NUM_REGIONS), jnp.float32),
        scratch_shapes=[
            pltpu.VMEM((ROWS, 128), jnp.float32),
            pltpu.VMEM((16, 128), jnp.float32),
            pltpu.VMEM((32, 128), jnp.float32),
        ],
        compiler_params=pltpu.CompilerParams(
            dimension_semantics=("arbitrary",),
            vmem_limit_bytes=100 * 1024 * 1024),
    )(periph_seq, imu3, traj3, w1p, w1i, w1t, b1r, W2, b2r)


def _topk_insert(ts, v):
    out = [jnp.maximum(ts[0], v)]
    for q in range(1, TOP_K):
        out.append(jnp.maximum(ts[q], jnp.minimum(ts[q - 1], v)))
    return tuple(out)


def _bfly_max(v):
    for k in range(4):
        perm = lax.iota(jnp.int32, LANES) ^ (1 << k)
        v = jnp.maximum(v, jnp.take(v, perm))
    return v


def _bfly_sum(v):
    for k in range(4):
        perm = lax.iota(jnp.int32, LANES) ^ (1 << k)
        v = v + jnp.take(v, perm)
    return v


def _kth_splat(vregs, k):
    zero = jnp.zeros((LANES,), jnp.float32)
    bound = jnp.full((LANES,), jnp.inf, jnp.float32)
    need = jnp.full((LANES,), float(k), jnp.float32)
    thresh = jnp.full((LANES,), BIG, jnp.float32)
    for _level in range(k):
        mm = jnp.full((LANES,), NEG, jnp.float32)
        for t in vregs:
            mm = jnp.maximum(mm, jnp.where(t < bound, t, NEG))
        m = _bfly_max(mm)
        cc = jnp.zeros((LANES,), jnp.float32)
        for t in vregs:
            cc += jnp.where(t == m, 1.0, 0.0)
        c = _bfly_sum(cc)
        tk = jnp.where(need > zero, 1.0, 0.0)
        mc = jnp.minimum(jnp.maximum(m, -BIG), BIG)
        thresh = jnp.minimum(thresh, tk * mc + (1.0 - tk) * BIG)
        need = need - c
        bound = m
    return thresh


CHUNK_SL = 8


def _row_thresh(buf, tsb):
    def p1(i, lm):
        base = i * (8 * LANES)
        vs = [buf[pl.ds(base + s * LANES, LANES)] for s in range(8)]
        m01 = jnp.maximum(vs[0], vs[1])
        m23 = jnp.maximum(vs[2], vs[3])
        m45 = jnp.maximum(vs[4], vs[5])
        m67 = jnp.maximum(vs[6], vs[7])
        return jnp.maximum(
            jnp.maximum(jnp.maximum(m01, m23), jnp.maximum(m45, m67)), lm)

    lm = lax.fori_loop(0, SLICES // 8, p1,
                       jnp.full((LANES,), NEG, jnp.float32))
    t0 = _kth_splat([lm], TOP_K)

    for q in range(TOP_K):
        tsb[pl.ds(q * LANES, LANES)] = jnp.full((LANES,), NEG, jnp.float32)

    def p2(i, carry):
        base = i * (CHUNK_SL * LANES)
        vs = [buf[pl.ds(base + s * LANES, LANES)] for s in range(CHUNK_SL)]
        m8 = vs[0]
        for s in range(1, CHUNK_SL):
            m8 = jnp.maximum(m8, vs[s])
        hasc = _bfly_max(jnp.where(m8 >= t0, 1.0, 0.0))

        @pl.when(hasc[0] > 0.5)
        def _ins():
            ts = tuple(tsb[pl.ds(q * LANES, LANES)] for q in range(TOP_K))
            for s in range(CHUNK_SL):
                ts = _topk_insert(ts, vs[s])
            for q in range(TOP_K):
                tsb[pl.ds(q * LANES, LANES)] = ts[q]

        return carry

    lax.fori_loop(0, SLICES // CHUNK_SL, p2, 0)
    ts = [tsb[pl.ds(q * LANES, LANES)] for q in range(TOP_K)]
    return _kth_splat(ts, TOP_K)


def _sc_gate_body(scores_hbm, gate_hbm, b0, b1, b2, tsb,
                  si0, si1, si2, so0, so1, so2):
    cid = lax.axis_index("c")
    sid = lax.axis_index("s")
    wid = sid * 2 + cid
    base_row = wid * 4
    bufs = [b0, b1, b2]
    isems = [si0, si1, si2]
    osems = [so0, so1, so2]

    inflight = {}
    for j in range(3):
        inflight[j] = pltpu.async_copy(
            scores_hbm.at[base_row + j], bufs[j], isems[j])
    outflight = {}
    for j in range(4):
        b = j % 3
        buf = bufs[b]
        inflight[j].wait()
        thresh = _row_thresh(buf, tsb)

        def p3(i, carry):
            base = i * (8 * LANES)
            for s in range(8):
                sl = pl.ds(base + s * LANES, LANES)
                v = buf[sl]
                buf[sl] = jnp.where(v >= thresh, 1.0, 0.0)
            return carry

        lax.fori_loop(0, SLICES // 8, p3, 0)
        outflight[j] = pltpu.async_copy(
            buf, gate_hbm.at[base_row + j], osems[b])
        if j == 0:
            outflight[0].wait()
            inflight[3] = pltpu.async_copy(
                scores_hbm.at[base_row + 3], bufs[0], isems[0])
    for j in range(1, 4):
        outflight[j].wait()


def _gate_sc(scores):
    mesh = plsc.VectorSubcoreMesh(core_axis_name="c", subcore_axis_name="s")
    f = functools.partial(
        pl.kernel,
        mesh=mesh,
        out_type=jax.ShapeDtypeStruct((ROWS, NUM_REGIONS), jnp.float32),
        scratch_types=[pltpu.VMEM((NUM_REGIONS,), jnp.float32),
                       pltpu.VMEM((NUM_REGIONS,), jnp.float32),
                       pltpu.VMEM((NUM_REGIONS,), jnp.float32),
                       pltpu.VMEM((TOP_K * LANES,), jnp.float32),
                       pltpu.SemaphoreType.DMA,
                       pltpu.SemaphoreType.DMA,
                       pltpu.SemaphoreType.DMA,
                       pltpu.SemaphoreType.DMA,
                       pltpu.SemaphoreType.DMA,
                       pltpu.SemaphoreType.DMA],
    )(_sc_gate_body)
    return f(scores)


@jax.jit
def kernel(periph_seq, imu_seq, traj_seq, W1, b1, W2, b2):
    imu3 = imu_seq.reshape(SEQ * 2048)
    traj3 = traj_seq.reshape(SEQ * 4096)
    w1p = W1[0:128]
    w1i = W1[128:144]
    w1t = W1[144:176]
    b1r = b1.reshape(1, HIDDEN)
    b2r = b2.reshape(1, NUM_REGIONS)
    scores = _scores_tc(periph_seq, imu3, traj3, w1p, w1i, w1t,
                        b1r, W2, b2r)
    gate = _gate_sc(scores)
    return (scores, gate)


# speedup vs baseline: 1.5391x; 1.5391x over previous
import functools

import jax
import jax.numpy as jnp
from jax import lax
from jax.experimental import pallas as pl
from jax.experimental.pallas import tpu as pltpu
from jax.experimental.pallas import tpu_sc as plsc

SEQ = 2048
ROWS = 128
NUM_REGIONS = 32768
HIDDEN = 64
TOP_K = 8
CHUNK = 64
LANES = 16
SLICES = NUM_REGIONS // LANES

NEG = float("-inf")
BIG = 3.0e38


def _mlp_body(periph_ref, imu_ref, traj_ref, w1p_ref, w1i_ref, w1t_ref,
              b1_ref, w2_ref, b2_ref, scores_ref, accp, acci, acct):
    g = pl.program_id(0)

    @pl.when(g == 0)
    def _init():
        accp[...] = jnp.zeros_like(accp)
        acci[...] = jnp.zeros_like(acci)
        acct[...] = jnp.zeros_like(acct)

    accp[...] += jnp.sum(periph_ref[...], axis=0)
    acci[...] += jnp.sum(imu_ref[...], axis=0)
    acct[...] += jnp.sum(traj_ref[...], axis=0)

    @pl.when(g == (SEQ // CHUNK) - 1)
    def _final():
        inv = jnp.float32(1.0 / SEQ)
        xi = (acci[...] * inv).reshape(ROWS, 16)
        xt = (acct[...] * inv).reshape(ROWS, 32)
        pre = (accp[...] * inv) @ w1p_ref[...]
        pre += xi @ w1i_ref[...]
        pre += xt @ w1t_ref[...]
        h = jnp.maximum(pre + b1_ref[...], 0.0)
        scores_ref[...] = h @ w2_ref[...] + b2_ref[...]


def _scores_tc(periph_seq, imu3, traj3, w1p, w1i, w1t, b1r, W2, b2r):
    nsteps = SEQ // CHUNK
    return pl.pallas_call(
        _mlp_body,
        grid=(nsteps,),
        in_specs=[
            pl.BlockSpec((CHUNK, ROWS, 128), lambda g: (g, 0, 0)),
            pl.BlockSpec((CHUNK, 16, 128), lambda g: (g, 0, 0)),
            pl.BlockSpec((CHUNK, 32, 128), lambda g: (g, 0, 0)),
            pl.BlockSpec((128, HIDDEN), lambda g: (0, 0)),
            pl.BlockSpec((16, HIDDEN), lambda g: (0, 0)),
            pl.BlockSpec((32, HIDDEN), lambda g: (0, 0)),
            pl.BlockSpec((1, HIDDEN), lambda g: (0, 0)),
            pl.BlockSpec((HIDDEN, NUM_REGIONS), lambda g: (0, 0)),
            pl.BlockSpec((1, NUM_REGIONS), lambda g: (0, 0)),
        ],
        out_specs=pl.BlockSpec((ROWS, NUM_REGIONS), lambda g: (0, 0)),
        out_shape=jax.ShapeDtypeStruct((ROWS, NUM_REGIONS), jnp.float32),
        scratch_shapes=[
            pltpu.VMEM((ROWS, 128), jnp.float32),
            pltpu.VMEM((16, 128), jnp.float32),
            pltpu.VMEM((32, 128), jnp.float32),
        ],
        compiler_params=pltpu.CompilerParams(
            dimension_semantics=("arbitrary",),
            vmem_limit_bytes=100 * 1024 * 1024),
    )(periph_seq, imu3, traj3, w1p, w1i, w1t, b1r, W2, b2r)


def _topk_insert(ts, v):
    out = [jnp.maximum(ts[0], v)]
    for q in range(1, TOP_K):
        out.append(jnp.maximum(ts[q], jnp.minimum(ts[q - 1], v)))
    return tuple(out)


def _bfly_max(v):
    for k in range(4):
        perm = lax.iota(jnp.int32, LANES) ^ (1 << k)
        v = jnp.maximum(v, jnp.take(v, perm))
    return v


def _bfly_sum(v):
    for k in range(4):
        perm = lax.iota(jnp.int32, LANES) ^ (1 << k)
        v = v + jnp.take(v, perm)
    return v


def _kth_splat(vregs, k):
    zero = jnp.zeros((LANES,), jnp.float32)
    bound = jnp.full((LANES,), jnp.inf, jnp.float32)
    need = jnp.full((LANES,), float(k), jnp.float32)
    thresh = jnp.full((LANES,), BIG, jnp.float32)
    for _level in range(k):
        mm = jnp.full((LANES,), NEG, jnp.float32)
        for t in vregs:
            mm = jnp.maximum(mm, jnp.where(t < bound, t, NEG))
        m = _bfly_max(mm)
        cc = jnp.zeros((LANES,), jnp.float32)
        for t in vregs:
            cc += jnp.where(t == m, 1.0, 0.0)
        c = _bfly_sum(cc)
        tk = jnp.where(need > zero, 1.0, 0.0)
        mc = jnp.minimum(jnp.maximum(m, -BIG), BIG)
        thresh = jnp.minimum(thresh, tk * mc + (1.0 - tk) * BIG)
        need = need - c
        bound = m
    return thresh


CHUNK_SL = 8


def _row_thresh(buf, tsb):
    def p1(i, lm):
        base = i * (8 * LANES)
        vs = [buf[pl.ds(base + s * LANES, LANES)] for s in range(8)]
        m01 = jnp.maximum(vs[0], vs[1])
        m23 = jnp.maximum(vs[2], vs[3])
        m45 = jnp.maximum(vs[4], vs[5])
        m67 = jnp.maximum(vs[6], vs[7])
        return jnp.maximum(
            jnp.maximum(jnp.maximum(m01, m23), jnp.maximum(m45, m67)), lm)

    lm = lax.fori_loop(0, SLICES // 8, p1,
                       jnp.full((LANES,), NEG, jnp.float32))
    t0 = _kth_splat([lm], TOP_K)

    for q in range(TOP_K):
        tsb[pl.ds(q * LANES, LANES)] = jnp.full((LANES,), NEG, jnp.float32)

    def p2(i, carry):
        base = i * (CHUNK_SL * LANES)
        vs = [buf[pl.ds(base + s * LANES, LANES)] for s in range(CHUNK_SL)]
        m8 = vs[0]
        for s in range(1, CHUNK_SL):
            m8 = jnp.maximum(m8, vs[s])
        hasc = _bfly_max(jnp.where(m8 >= t0, 1.0, 0.0))

        @pl.when(hasc[0] > 0.5)
        def _ins():
            ts = tuple(tsb[pl.ds(q * LANES, LANES)] for q in range(TOP_K))
            for s in range(CHUNK_SL):
                ts = _topk_insert(ts, vs[s])
            for q in range(TOP_K):
                tsb[pl.ds(q * LANES, LANES)] = ts[q]

        return carry

    lax.fori_loop(0, SLICES // CHUNK_SL, p2, 0)
    ts = [tsb[pl.ds(q * LANES, LANES)] for q in range(TOP_K)]
    return _kth_splat(ts, TOP_K)


def _sc_gate_body(scores_hbm, gate_hbm, b0, b1, b2, tsb,
                  si0, si1, si2, so0, so1, so2):
    cid = lax.axis_index("c")
    sid = lax.axis_index("s")
    wid = sid * 2 + cid
    base_row = wid * 4
    bufs = [b0, b1, b2]
    isems = [si0, si1, si2]
    osems = [so0, so1, so2]

    inflight = {}
    for j in range(3):
        inflight[j] = pltpu.async_copy(
            scores_hbm.at[base_row + j], bufs[j], isems[j])
    outflight = {}
    for j in range(4):
        b = j % 3
        buf = bufs[b]
        inflight[j].wait()
        thresh = _row_thresh(buf, tsb)

        def p3(i, carry):
            base = i * (8 * LANES)
            for s in range(8):
                sl = pl.ds(base + s * LANES, LANES)
                v = buf[sl]
                buf[sl] = jnp.where(v >= thresh, 1.0, 0.0)
            return carry

        lax.fori_loop(0, SLICES // 8, p3, 0)
        outflight[j] = pltpu.async_copy(
            buf, gate_hbm.at[base_row + j], osems[b])
        if j == 0:
            outflight[0].wait()
            inflight[3] = pltpu.async_copy(
                scores_hbm.at[base_row + 3], bufs[0], isems[0])
    for j in range(1, 4):
        outflight[j].wait()


def _gate_sc(scores):
    mesh = plsc.VectorSubcoreMesh(core_axis_name="c", subcore_axis_name="s")
    f = functools.partial(
        pl.kernel,
        mesh=mesh,
        out_type=jax.ShapeDtypeStruct((ROWS, NUM_REGIONS), jnp.float32),
        scratch_types=[pltpu.VMEM((NUM_REGIONS,), jnp.float32),
                       pltpu.VMEM((NUM_REGIONS,), jnp.float32),
                       pltpu.VMEM((NUM_REGIONS,), jnp.float32),
                       pltpu.VMEM((TOP_K * LANES,), jnp.float32),
                       pltpu.SemaphoreType.DMA,
                       pltpu.SemaphoreType.DMA,
                       pltpu.SemaphoreType.DMA,
                       pltpu.SemaphoreType.DMA,
                       pltpu.SemaphoreType.DMA,
                       pltpu.SemaphoreType.DMA],
    )(_sc_gate_body)
    return f(scores)


@jax.jit
def kernel(periph_seq, imu_seq, traj_seq, W1, b1, W2, b2):
    imu3 = imu_seq.reshape(SEQ, 16, 128)
    traj3 = traj_seq.reshape(SEQ, 32, 128)
    w1p = W1[0:128]
    w1i = W1[128:144]
    w1t = W1[144:176]
    b1r = b1.reshape(1, HIDDEN)
    b2r = b2.reshape(1, NUM_REGIONS)
    scores = _scores_tc(periph_seq, imu3, traj3, w1p, w1i, w1t,
                        b1r, W2, b2r)
    gate = _gate_sc(scores)
    return (scores, gate)


# speedup vs baseline: 1.5491x; 1.0064x over previous
import functools

import jax
import jax.numpy as jnp
from jax import lax
from jax.experimental import pallas as pl
from jax.experimental.pallas import tpu as pltpu
from jax.experimental.pallas import tpu_sc as plsc

SEQ = 2048
ROWS = 128
NUM_REGIONS = 32768
HIDDEN = 64
TOP_K = 8
CHUNK = 128
LANES = 16
SLICES = NUM_REGIONS // LANES

NEG = float("-inf")
BIG = 3.0e38


def _mlp_body(periph_ref, imu_ref, traj_ref, w1p_ref, w1i_ref, w1t_ref,
              b1_ref, w2_ref, b2_ref, scores_ref, accp, acci, acct):
    g = pl.program_id(0)

    @pl.when(g == 0)
    def _init():
        accp[...] = jnp.zeros_like(accp)
        acci[...] = jnp.zeros_like(acci)
        acct[...] = jnp.zeros_like(acct)

    accp[...] += jnp.sum(periph_ref[...], axis=0)
    acci[...] += jnp.sum(imu_ref[...], axis=0)
    acct[...] += jnp.sum(traj_ref[...], axis=0)

    @pl.when(g == (SEQ // CHUNK) - 1)
    def _final():
        inv = jnp.float32(1.0 / SEQ)
        xi = (acci[...] * inv).reshape(ROWS, 16)
        xt = (acct[...] * inv).reshape(ROWS, 32)
        pre = (accp[...] * inv) @ w1p_ref[...]
        pre += xi @ w1i_ref[...]
        pre += xt @ w1t_ref[...]
        h = jnp.maximum(pre + b1_ref[...], 0.0)
        scores_ref[...] = h @ w2_ref[...] + b2_ref[...]


def _scores_tc(periph_seq, imu3, traj3, w1p, w1i, w1t, b1r, W2, b2r):
    nsteps = SEQ // CHUNK
    return pl.pallas_call(
        _mlp_body,
        grid=(nsteps,),
        in_specs=[
            pl.BlockSpec((CHUNK, ROWS, 128), lambda g: (g, 0, 0)),
            pl.BlockSpec((CHUNK, 16, 128), lambda g: (g, 0, 0)),
            pl.BlockSpec((CHUNK, 32, 128), lambda g: (g, 0, 0)),
            pl.BlockSpec((128, HIDDEN), lambda g: (0, 0)),
            pl.BlockSpec((16, HIDDEN), lambda g: (0, 0)),
            pl.BlockSpec((32, HIDDEN), lambda g: (0, 0)),
            pl.BlockSpec((1, HIDDEN), lambda g: (0, 0)),
            pl.BlockSpec((HIDDEN, NUM_REGIONS), lambda g: (0, 0)),
            pl.BlockSpec((1, NUM_REGIONS), lambda g: (0, 0)),
        ],
        out_specs=pl.BlockSpec((ROWS, NUM_REGIONS), lambda g: (0, 0)),
        out_shape=jax.ShapeDtypeStruct((ROWS, NUM_REGIONS), jnp.float32),
        scratch_shapes=[
            pltpu.VMEM((ROWS, 128), jnp.float32),
            pltpu.VMEM((16, 128), jnp.float32),
            pltpu.VMEM((32, 128), jnp.float32),
        ],
        compiler_params=pltpu.CompilerParams(
            dimension_semantics=("arbitrary",),
            vmem_limit_bytes=100 * 1024 * 1024),
    )(periph_seq, imu3, traj3, w1p, w1i, w1t, b1r, W2, b2r)


def _topk_insert(ts, v):
    out = [jnp.maximum(ts[0], v)]
    for q in range(1, TOP_K):
        out.append(jnp.maximum(ts[q], jnp.minimum(ts[q - 1], v)))
    return tuple(out)


def _bfly_max(v):
    for k in range(4):
        perm = lax.iota(jnp.int32, LANES) ^ (1 << k)
        v = jnp.maximum(v, jnp.take(v, perm))
    return v


def _bfly_sum(v):
    for k in range(4):
        perm = lax.iota(jnp.int32, LANES) ^ (1 << k)
        v = v + jnp.take(v, perm)
    return v


def _kth_splat(vregs, k):
    zero = jnp.zeros((LANES,), jnp.float32)
    bound = jnp.full((LANES,), jnp.inf, jnp.float32)
    need = jnp.full((LANES,), float(k), jnp.float32)
    thresh = jnp.full((LANES,), BIG, jnp.float32)
    for _level in range(k):
        mm = jnp.full((LANES,), NEG, jnp.float32)
        for t in vregs:
            mm = jnp.maximum(mm, jnp.where(t < bound, t, NEG))
        m = _bfly_max(mm)
        cc = jnp.zeros((LANES,), jnp.float32)
        for t in vregs:
            cc += jnp.where(t == m, 1.0, 0.0)
        c = _bfly_sum(cc)
        tk = jnp.where(need > zero, 1.0, 0.0)
        mc = jnp.minimum(jnp.maximum(m, -BIG), BIG)
        thresh = jnp.minimum(thresh, tk * mc + (1.0 - tk) * BIG)
        need = need - c
        bound = m
    return thresh


CHUNK_SL = 8


def _row_thresh(buf, tsb):
    def p1(i, lm):
        base = i * (8 * LANES)
        vs = [buf[pl.ds(base + s * LANES, LANES)] for s in range(8)]
        m01 = jnp.maximum(vs[0], vs[1])
        m23 = jnp.maximum(vs[2], vs[3])
        m45 = jnp.maximum(vs[4], vs[5])
        m67 = jnp.maximum(vs[6], vs[7])
        return jnp.maximum(
            jnp.maximum(jnp.maximum(m01, m23), jnp.maximum(m45, m67)), lm)

    lm = lax.fori_loop(0, SLICES // 8, p1,
                       jnp.full((LANES,), NEG, jnp.float32))
    t0 = _kth_splat([lm], TOP_K)

    for q in range(TOP_K):
        tsb[pl.ds(q * LANES, LANES)] = jnp.full((LANES,), NEG, jnp.float32)

    def p2(i, carry):
        base = i * (CHUNK_SL * LANES)
        vs = [buf[pl.ds(base + s * LANES, LANES)] for s in range(CHUNK_SL)]
        m8 = vs[0]
        for s in range(1, CHUNK_SL):
            m8 = jnp.maximum(m8, vs[s])
        hasc = _bfly_max(jnp.where(m8 >= t0, 1.0, 0.0))

        @pl.when(hasc[0] > 0.5)
        def _ins():
            ts = tuple(tsb[pl.ds(q * LANES, LANES)] for q in range(TOP_K))
            for s in range(CHUNK_SL):
                ts = _topk_insert(ts, vs[s])
            for q in range(TOP_K):
                tsb[pl.ds(q * LANES, LANES)] = ts[q]

        return carry

    lax.fori_loop(0, SLICES // CHUNK_SL, p2, 0)
    ts = [tsb[pl.ds(q * LANES, LANES)] for q in range(TOP_K)]
    return _kth_splat(ts, TOP_K)


def _sc_gate_body(scores_hbm, gate_hbm, b0, b1, b2, tsb,
                  si0, si1, si2, so0, so1, so2):
    cid = lax.axis_index("c")
    sid = lax.axis_index("s")
    wid = sid * 2 + cid
    base_row = wid * 4
    bufs = [b0, b1, b2]
    isems = [si0, si1, si2]
    osems = [so0, so1, so2]

    inflight = {}
    for j in range(3):
        inflight[j] = pltpu.async_copy(
            scores_hbm.at[base_row + j], bufs[j], isems[j])
    outflight = {}
    for j in range(4):
        b = j % 3
        buf = bufs[b]
        inflight[j].wait()
        thresh = _row_thresh(buf, tsb)

        def p3(i, carry):
            base = i * (8 * LANES)
            for s in range(8):
                sl = pl.ds(base + s * LANES, LANES)
                v = buf[sl]
                buf[sl] = jnp.where(v >= thresh, 1.0, 0.0)
            return carry

        lax.fori_loop(0, SLICES // 8, p3, 0)
        outflight[j] = pltpu.async_copy(
            buf, gate_hbm.at[base_row + j], osems[b])
        if j == 0:
            outflight[0].wait()
            inflight[3] = pltpu.async_copy(
                scores_hbm.at[base_row + 3], bufs[0], isems[0])
    for j in range(1, 4):
        outflight[j].wait()


def _gate_sc(scores):
    mesh = plsc.VectorSubcoreMesh(core_axis_name="c", subcore_axis_name="s")
    f = functools.partial(
        pl.kernel,
        mesh=mesh,
        out_type=jax.ShapeDtypeStruct((ROWS, NUM_REGIONS), jnp.float32),
        scratch_types=[pltpu.VMEM((NUM_REGIONS,), jnp.float32),
                       pltpu.VMEM((NUM_REGIONS,), jnp.float32),
                       pltpu.VMEM((NUM_REGIONS,), jnp.float32),
                       pltpu.VMEM((TOP_K * LANES,), jnp.float32),
                       pltpu.SemaphoreType.DMA,
                       pltpu.SemaphoreType.DMA,
                       pltpu.SemaphoreType.DMA,
                       pltpu.SemaphoreType.DMA,
                       pltpu.SemaphoreType.DMA,
                       pltpu.SemaphoreType.DMA],
    )(_sc_gate_body)
    return f(scores)


@jax.jit
def kernel(periph_seq, imu_seq, traj_seq, W1, b1, W2, b2):
    imu3 = imu_seq.reshape(SEQ, 16, 128)
    traj3 = traj_seq.reshape(SEQ, 32, 128)
    w1p = W1[0:128]
    w1i = W1[128:144]
    w1t = W1[144:176]
    b1r = b1.reshape(1, HIDDEN)
    b2r = b2.reshape(1, NUM_REGIONS)
    scores = _scores_tc(periph_seq, imu3, traj3, w1p, w1i, w1t,
                        b1r, W2, b2r)
    gate = _gate_sc(scores)
    return (scores, gate)


# speedup vs baseline: 1.6673x; 1.0764x over previous
import functools

import jax
import jax.numpy as jnp
from jax import lax
from jax.experimental import pallas as pl
from jax.experimental.pallas import tpu as pltpu
from jax.experimental.pallas import tpu_sc as plsc

SEQ = 2048
ROWS = 128
NUM_REGIONS = 32768
HIDDEN = 64
TOP_K = 8
CHUNK = 128
LANES = 16
SLICES = NUM_REGIONS // LANES

NEG = float("-inf")
BIG = 3.0e38


def _mlp_body(periph_ref, imu_ref, traj_ref, w1p_ref, w1i_ref, w1t_ref,
              b1_ref, w2_ref, b2_ref, scores_ref, accp, acci, acct):
    g = pl.program_id(0)

    @pl.when(g == 0)
    def _init():
        accp[...] = jnp.zeros_like(accp)
        acci[...] = jnp.zeros_like(acci)
        acct[...] = jnp.zeros_like(acct)

    accp[...] += jnp.sum(periph_ref[...], axis=0)
    acci[...] += jnp.sum(imu_ref[...], axis=0)
    acct[...] += jnp.sum(traj_ref[...], axis=0)

    @pl.when(g == (SEQ // CHUNK) - 1)
    def _final():
        inv = jnp.float32(1.0 / SEQ)
        xi = (acci[...] * inv).reshape(ROWS, 16)
        xt = (acct[...] * inv).reshape(ROWS, 32)
        pre = (accp[...] * inv) @ w1p_ref[...]
        pre += xi @ w1i_ref[...]
        pre += xt @ w1t_ref[...]
        h = jnp.maximum(pre + b1_ref[...], 0.0)
        scores_ref[...] = h @ w2_ref[...] + b2_ref[...]


def _scores_tc(periph_seq, imu3, traj3, w1p, w1i, w1t, b1r, W2, b2r):
    nsteps = SEQ // CHUNK
    return pl.pallas_call(
        _mlp_body,
        grid=(nsteps,),
        in_specs=[
            pl.BlockSpec((CHUNK, ROWS, 128), lambda g: (g, 0, 0)),
            pl.BlockSpec((CHUNK, 16, 128), lambda g: (g, 0, 0)),
            pl.BlockSpec((CHUNK, 32, 128), lambda g: (g, 0, 0)),
            pl.BlockSpec((128, HIDDEN), lambda g: (0, 0)),
            pl.BlockSpec((16, HIDDEN), lambda g: (0, 0)),
            pl.BlockSpec((32, HIDDEN), lambda g: (0, 0)),
            pl.BlockSpec((1, HIDDEN), lambda g: (0, 0)),
            pl.BlockSpec((HIDDEN, NUM_REGIONS), lambda g: (0, 0)),
            pl.BlockSpec((1, NUM_REGIONS), lambda g: (0, 0)),
        ],
        out_specs=pl.BlockSpec((ROWS, NUM_REGIONS), lambda g: (0, 0)),
        out_shape=jax.ShapeDtypeStruct((ROWS, NUM_REGIONS), jnp.float32),
        scratch_shapes=[
            pltpu.VMEM((ROWS, 128), jnp.float32),
            pltpu.VMEM((16, 128), jnp.float32),
            pltpu.VMEM((32, 128), jnp.float32),
        ],
        compiler_params=pltpu.CompilerParams(
            dimension_semantics=("arbitrary",),
            vmem_limit_bytes=100 * 1024 * 1024),
    )(periph_seq, imu3, traj3, w1p, w1i, w1t, b1r, W2, b2r)


def _topk_insert(ts, v):
    out = [jnp.maximum(ts[0], v)]
    for q in range(1, TOP_K):
        out.append(jnp.maximum(ts[q], jnp.minimum(ts[q - 1], v)))
    return tuple(out)


def _bfly_max(v):
    for k in range(4):
        perm = lax.iota(jnp.int32, LANES) ^ (1 << k)
        v = jnp.maximum(v, jnp.take(v, perm))
    return v


def _bfly_sum(v):
    for k in range(4):
        perm = lax.iota(jnp.int32, LANES) ^ (1 << k)
        v = v + jnp.take(v, perm)
    return v


def _kth_splat(vregs, k):
    zero = jnp.zeros((LANES,), jnp.float32)
    bound = jnp.full((LANES,), jnp.inf, jnp.float32)
    need = jnp.full((LANES,), float(k), jnp.float32)
    thresh = jnp.full((LANES,), BIG, jnp.float32)
    for _level in range(k):
        mm = jnp.full((LANES,), NEG, jnp.float32)
        for t in vregs:
            mm = jnp.maximum(mm, jnp.where(t < bound, t, NEG))
        m = _bfly_max(mm)
        cc = jnp.zeros((LANES,), jnp.float32)
        for t in vregs:
            cc += jnp.where(t == m, 1.0, 0.0)
        c = _bfly_sum(cc)
        tk = jnp.where(need > zero, 1.0, 0.0)
        mc = jnp.minimum(jnp.maximum(m, -BIG), BIG)
        thresh = jnp.minimum(thresh, tk * mc + (1.0 - tk) * BIG)
        need = need - c
        bound = m
    return thresh


CHUNK_SL = 32


def _row_thresh(buf, tsb):
    def p1(i, lm):
        base = i * (8 * LANES)
        vs = [buf[pl.ds(base + s * LANES, LANES)] for s in range(8)]
        m01 = jnp.maximum(vs[0], vs[1])
        m23 = jnp.maximum(vs[2], vs[3])
        m45 = jnp.maximum(vs[4], vs[5])
        m67 = jnp.maximum(vs[6], vs[7])
        return jnp.maximum(
            jnp.maximum(jnp.maximum(m01, m23), jnp.maximum(m45, m67)), lm)

    lm = lax.fori_loop(0, SLICES // 8, p1,
                       jnp.full((LANES,), NEG, jnp.float32))
    t0 = _kth_splat([lm], TOP_K)

    for q in range(TOP_K):
        tsb[pl.ds(q * LANES, LANES)] = jnp.full((LANES,), NEG, jnp.float32)

    def p2(i, carry):
        base = i * (CHUNK_SL * LANES)
        vs = [buf[pl.ds(base + s * LANES, LANES)] for s in range(CHUNK_SL)]
        m8 = vs[0]
        for s in range(1, CHUNK_SL):
            m8 = jnp.maximum(m8, vs[s])
        hasc = _bfly_max(jnp.where(m8 >= t0, 1.0, 0.0))

        @pl.when(hasc[0] > 0.5)
        def _ins():
            ts = tuple(tsb[pl.ds(q * LANES, LANES)] for q in range(TOP_K))
            for s in range(CHUNK_SL):
                ts = _topk_insert(ts, vs[s])
            for q in range(TOP_K):
                tsb[pl.ds(q * LANES, LANES)] = ts[q]

        return carry

    lax.fori_loop(0, SLICES // CHUNK_SL, p2, 0)
    ts = [tsb[pl.ds(q * LANES, LANES)] for q in range(TOP_K)]
    return _kth_splat(ts, TOP_K)


def _sc_gate_body(scores_hbm, gate_hbm, b0, b1, b2, tsb,
                  si0, si1, si2, so0, so1, so2):
    cid = lax.axis_index("c")
    sid = lax.axis_index("s")
    wid = sid * 2 + cid
    base_row = wid * 4
    bufs = [b0, b1, b2]
    isems = [si0, si1, si2]
    osems = [so0, so1, so2]

    inflight = {}
    for j in range(3):
        inflight[j] = pltpu.async_copy(
            scores_hbm.at[base_row + j], bufs[j], isems[j])
    outflight = {}
    for j in range(4):
        b = j % 3
        buf = bufs[b]
        inflight[j].wait()
        thresh = _row_thresh(buf, tsb)

        def p3(i, carry):
            base = i * (8 * LANES)
            for s in range(8):
                sl = pl.ds(base + s * LANES, LANES)
                v = buf[sl]
                buf[sl] = jnp.where(v >= thresh, 1.0, 0.0)
            return carry

        lax.fori_loop(0, SLICES // 8, p3, 0)
        outflight[j] = pltpu.async_copy(
            buf, gate_hbm.at[base_row + j], osems[b])
        if j == 0:
            outflight[0].wait()
            inflight[3] = pltpu.async_copy(
                scores_hbm.at[base_row + 3], bufs[0], isems[0])
    for j in range(1, 4):
        outflight[j].wait()


def _gate_sc(scores):
    mesh = plsc.VectorSubcoreMesh(core_axis_name="c", subcore_axis_name="s")
    f = functools.partial(
        pl.kernel,
        mesh=mesh,
        out_type=jax.ShapeDtypeStruct((ROWS, NUM_REGIONS), jnp.float32),
        scratch_types=[pltpu.VMEM((NUM_REGIONS,), jnp.float32),
                       pltpu.VMEM((NUM_REGIONS,), jnp.float32),
                       pltpu.VMEM((NUM_REGIONS,), jnp.float32),
                       pltpu.VMEM((TOP_K * LANES,), jnp.float32),
                       pltpu.SemaphoreType.DMA,
                       pltpu.SemaphoreType.DMA,
                       pltpu.SemaphoreType.DMA,
                       pltpu.SemaphoreType.DMA,
                       pltpu.SemaphoreType.DMA,
                       pltpu.SemaphoreType.DMA],
    )(_sc_gate_body)
    return f(scores)


@jax.jit
def kernel(periph_seq, imu_seq, traj_seq, W1, b1, W2, b2):
    imu3 = imu_seq.reshape(SEQ, 16, 128)
    traj3 = traj_seq.reshape(SEQ, 32, 128)
    w1p = W1[0:128]
    w1i = W1[128:144]
    w1t = W1[144:176]
    b1r = b1.reshape(1, HIDDEN)
    b2r = b2.reshape(1, NUM_REGIONS)
    scores = _scores_tc(periph_seq, imu3, traj3, w1p, w1i, w1t,
                        b1r, W2, b2r)
    gate = _gate_sc(scores)
    return (scores, gate)
